# baseline (speedup 1.0000x reference)
import jax
import jax.numpy as jnp
from jax import lax
from jax.experimental import pallas as pl
from jax.experimental.pallas import tpu as pltpu

N = 100
NP = 128
IN_DIM = 512
HID = 512
H = 8
D = HID * H
FF = HID * 4
OUT_DIM = 256
K = 10
NEG = -1e30
F32 = jnp.float32

_CONTRACT_11 = (((1,), (1,)), ((), ()))
_CONTRACT_10 = (((1,), (0,)), ((), ()))

_NSLOT = 5
_B_W1, _B_W2, _B_QKV, _B_PROJ, _B_FF1, _B_FF2, _B_OUTP = 1, 9, 17, 41, 49, 53, 61
_NBLK = 62


def _dot(a, b, dims):
    return lax.dot_general(a, b, dims, preferred_element_type=F32)


def _ln(xa):
    mu = jnp.mean(xa, axis=1, keepdims=True)
    var = jnp.mean((xa - mu) ** 2, axis=1, keepdims=True)
    return (xa - mu) / jnp.sqrt(var + 1e-5)


def _graph_setup(x, stats_ref, cs):
    sims = _dot(x, x, _CONTRACT_11)
    col = lax.broadcasted_iota(jnp.int32, (NP, NP), 1)
    row = lax.broadcasted_iota(jnp.int32, (NP, NP), 0)
    valid_col = col < N
    valid_row = row < N

    selected = jnp.zeros((NP, NP), jnp.bool_)
    t_mat = jnp.zeros((NP, NP), F32)
    for t in range(K + 1):
        masked = jnp.where(valid_col & (~selected), sims, NEG)
        rowmax = jnp.max(masked, axis=1, keepdims=True)
        cand = jnp.where(masked == rowmax, col, NP)
        first = jnp.min(cand, axis=1, keepdims=True)
        newsel = col == first
        selected = selected | newsel
        if t > 0:
            t_mat = t_mat + newsel.astype(F32)
    t_mat = jnp.where(valid_row, t_mat, 0.0)
    cs[...] = t_mat.T + jnp.where((row == col) & valid_row, 1.0, 0.0)

    centrality = jnp.sum(sims, axis=1, keepdims=True)
    validr1 = lax.broadcasted_iota(jnp.int32, (NP, 1), 0) < N
    cmean = jnp.sum(centrality) / N
    cvar = jnp.sum(jnp.where(validr1, (centrality - cmean) ** 2, 0.0)) / (N - 1)
    cstd = jnp.sqrt(cvar)
    smean = jnp.sum(sims) / (N * N)
    degree = jnp.sum((sims > 0.5).astype(F32), axis=1, keepdims=True)
    s2 = _dot(sims, sims, _CONTRACT_10)
    tri = jnp.sum(s2 * sims, axis=1, keepdims=True)
    clus = tri / (degree * (degree - 1.0) + 1e-8)
    clustering = jnp.sum(jnp.where(validr1, clus, 0.0)) / N

    lane = lax.broadcasted_iota(jnp.int32, (8, 128), 1)
    rw = lax.broadcasted_iota(jnp.int32, (8, 128), 0)
    stats_ref[...] = jnp.where((rw == 0) & (lane == 0), cmean,
                     jnp.where((rw == 0) & (lane == 1), clustering,
                     jnp.where((rw == 0) & (lane == 2), smean,
                     jnp.where((rw == 0) & (lane == 3), cstd, 0.0))))


def _gat_attend(xp, asrc_ref, adst_ref, cmat, h):
    asr = asrc_ref[pl.ds(h, 1), :]
    adr = adst_ref[pl.ds(h, 1), :]
    a_s_row = _dot(asr, xp, _CONTRACT_11)
    a_d_col = _dot(xp, adr, _CONTRACT_11)
    e = a_s_row + a_d_col
    e = jnp.where(e >= 0, e, 0.2 * e)
    mask = cmat > 0.0
    em = jnp.where(mask, e, NEG)
    m = jnp.max(em, axis=1, keepdims=True)
    m = jnp.where(m > 0.5 * NEG, m, 0.0)
    ex = jnp.where(mask, jnp.exp(e - m), 0.0)
    z = jnp.sum(cmat * ex, axis=1, keepdims=True)
    alpha = ex / (z + 1e-16)
    out = _dot(cmat * alpha, xp, _CONTRACT_10)
    out = jnp.where(out > 0, out, jnp.exp(out) - 1.0)
    validr = lax.broadcasted_iota(jnp.int32, (NP, HID), 0) < N
    return jnp.where(validr, out, 0.0), alpha


def _attn_head(qkvs, asc, h):
    qh = qkvs[:, pl.ds(h * HID, HID)]
    kh = qkvs[:, pl.ds((h + H) * HID, HID)]
    vh = qkvs[:, pl.ds((h + 2 * H) * HID, HID)]
    logits = _dot(qh, kh, _CONTRACT_11) * (1.0 / jnp.sqrt(HID * 1.0))
    colmask = lax.broadcasted_iota(jnp.int32, (NP, NP), 1) < N
    logits = jnp.where(colmask, logits, NEG)
    m = jnp.max(logits, axis=1, keepdims=True)
    e = jnp.exp(logits - m)
    e = jnp.where(colmask, e, 0.0)
    att = e / jnp.sum(e, axis=1, keepdims=True)
    asc[:, pl.ds(h * HID, HID)] = _dot(att, vh, _CONTRACT_10)


def _edge_entropy(v, cmat, mask):
    vm = jnp.where(mask, v, NEG)
    mx = jnp.max(vm)
    e = jnp.where(mask, jnp.exp(v - mx), 0.0)
    s = jnp.sum(cmat * e)
    pr = e / s
    term = jnp.where(mask, pr * jnp.log(pr + 1e-8), 0.0)
    return -jnp.sum(cmat * term)


def _mega_body(x_ref, pe_ref, w0_ref, w1_ref, w2_ref,
               as0_ref, ad0_ref, as1_ref, ad1_ref, as2_ref, ad2_ref,
               win_ref, wout_ref, wf1_ref, wf2_ref, wo_ref,
               out_ref, stats_ref,
               g0s, gas, gbs, xpa, qkvs, asc, accs, x1s, fs, cs, vs,
               slots, sems):
    def copy(b):
        s = b % _NSLOT
        if b == 0:
            return pltpu.make_async_copy(w0_ref, slots.at[s], sems.at[s])
        if b < _B_W2:
            src = w1_ref.at[pl.ds((b - _B_W1) * HID, HID), :]
        elif b < _B_QKV:
            src = w2_ref.at[pl.ds((b - _B_W2) * HID, HID), :]
        elif b < _B_PROJ:
            src = win_ref.at[pl.ds((b - _B_QKV) * HID, HID), :]
        elif b < _B_FF1:
            src = wout_ref.at[pl.ds((b - _B_PROJ) * HID, HID), :]
        elif b < _B_FF2:
            src = wf1_ref.at[pl.ds((b - _B_FF1) * HID, HID), :]
        elif b < _B_OUTP:
            return pltpu.make_async_copy(
                wf2_ref.at[pl.ds((b - _B_FF2) * HID, HID), :],
                slots.at[s, :, pl.ds(0, FF)], sems.at[s])
        else:
            return pltpu.make_async_copy(
                wo_ref, slots.at[s, pl.ds(0, OUT_DIM), :], sems.at[s])
        return pltpu.make_async_copy(src, slots.at[s], sems.at[s])

    for b in range(_NSLOT - 1):
        copy(b).start()

    x = x_ref[...]
    _graph_setup(x, stats_ref, cs)
    validg = lax.broadcasted_iota(jnp.int32, (NP, IN_DIM), 0) < N
    g0s[...] = jnp.where(validg, x + pe_ref[...], 0.0)

    def gat_attn_all(asrc_ref, adst_ref, gout, lidx):
        vsum = None
        for h in range(H):
            xph = xpa[:, pl.ds(h * HID, HID)]
            out, alpha = _gat_attend(xph, asrc_ref, adst_ref, cs[...], h)
            gout[:, pl.ds(h * HID, HID)] = out
            vsum = alpha * (1.0 / H) if vsum is None else vsum + alpha * (1.0 / H)
        vs[lidx] = vsum

    for b in range(_NBLK):
        copy(b).wait()
        if b + _NSLOT - 1 < _NBLK:
            copy(b + _NSLOT - 1).start()
        s = b % _NSLOT
        w = slots[s]
        if b == 0:
            xpa[...] = _dot(g0s[...], w, _CONTRACT_10)
            gat_attn_all(as0_ref, ad0_ref, gas, 0)
        elif b < _B_W2:
            r = b - _B_W1
            part = _dot(gas[:, pl.ds(r * HID, HID)], w, _CONTRACT_10)
            if r == 0:
                xpa[...] = part
            else:
                xpa[...] += part
            if r == H - 1:
                gat_attn_all(as1_ref, ad1_ref, gbs, 1)
        elif b < _B_QKV:
            r = b - _B_W2
            part = _dot(gbs[:, pl.ds(r * HID, HID)], w, _CONTRACT_10)
            if r == 0:
                xpa[...] = part
            else:
                xpa[...] += part
            if r == H - 1:
                gat_attn_all(as2_ref, ad2_ref, gas, 2)
        elif b < _B_PROJ:
            r = b - _B_QKV
            qkvs[:, pl.ds(r * HID, HID)] = _dot(gas[...], w, _CONTRACT_11)
            if r >= 2 * H + 1:
                _attn_head(qkvs, asc, r - (2 * H + 1))
        elif b < _B_FF1:
            jj = b - _B_PROJ
            if jj == 0:
                _attn_head(qkvs, asc, H - 1)
            accs[:, pl.ds(jj * HID, HID)] = _dot(asc[...], w, _CONTRACT_11)
            if jj == H - 1:
                x1s[...] = _ln(gas[...] + accs[...])
        elif b < _B_FF2:
            r = b - _B_FF1
            part = _dot(x1s[...], w, _CONTRACT_11)
            fs[:, pl.ds(r * HID, HID)] = jnp.maximum(part, 0.0)
        elif b < _B_OUTP:
            jj = b - _B_FF2
            accs[:, pl.ds(jj * HID, HID)] = _dot(fs[...], slots[s][:, :FF],
                                                 _CONTRACT_11)
        else:
            t = _ln(x1s[...] + accs[...])
            validr = lax.broadcasted_iota(jnp.int32, (NP, D), 0) < N
            tmean = jnp.sum(jnp.where(validr, t, 0.0), axis=0,
                            keepdims=True) / N
            out = _dot(tmean, slots[s][:OUT_DIM, :], _CONTRACT_11)
            out_ref[...] = jnp.broadcast_to(out, (8, OUT_DIM))

            cmat = cs[...]
            mask = cmat > 0.0
            ent = (_edge_entropy(vs[0], cmat, mask)
                   + _edge_entropy(vs[1], cmat, mask)
                   + _edge_entropy(vs[2], cmat, mask)) / 3.0
            lane = lax.broadcasted_iota(jnp.int32, (8, 128), 1)
            rw = lax.broadcasted_iota(jnp.int32, (8, 128), 0)
            stats_ref[...] = stats_ref[...] + jnp.where(
                (rw == 0) & (lane == 4), ent, 0.0)


def kernel(x, params):
    xp = jnp.pad(x, ((0, NP - N), (0, 0)))
    pep = jnp.pad(params['topo_pe'][:N, :IN_DIM], ((0, NP - N), (0, 0)))
    gat = params['gat']

    vm = pl.BlockSpec(memory_space=pltpu.VMEM)
    anym = pl.BlockSpec(memory_space=pl.ANY)
    outr, stats = pl.pallas_call(
        _mega_body,
        in_specs=[vm, vm, anym, anym, anym, vm, vm, vm, vm, vm, vm,
                  anym, anym, anym, anym, anym],
        out_specs=(vm, vm),
        out_shape=(
            jax.ShapeDtypeStruct((8, OUT_DIM), F32),
            jax.ShapeDtypeStruct((8, 128), F32),
        ),
        scratch_shapes=[
            pltpu.VMEM((NP, IN_DIM), F32),
            pltpu.VMEM((NP, D), F32),
            pltpu.VMEM((NP, D), F32),
            pltpu.VMEM((NP, D), F32),
            pltpu.VMEM((NP, 3 * D), F32),
            pltpu.VMEM((NP, D), F32),
            pltpu.VMEM((NP, D), F32),
            pltpu.VMEM((NP, D), F32),
            pltpu.VMEM((NP, FF), F32),
            pltpu.VMEM((NP, NP), F32),
            pltpu.VMEM((3, NP, NP), F32),
            pltpu.VMEM((_NSLOT, HID, D), F32),
            pltpu.SemaphoreType.DMA((_NSLOT,)),
        ],
    )(xp, pep, gat[0]['W'], gat[1]['W'], gat[2]['W'],
      gat[0]['a_src'], gat[0]['a_dst'], gat[1]['a_src'], gat[1]['a_dst'],
      gat[2]['a_src'], gat[2]['a_dst'],
      params['in_proj_w'], params['out_proj_w'],
      params['ff1_w'], params['ff2_w'], params['outp_w'])

    out = outr[0]
    return (out, stats[0, 0], stats[0, 1], stats[0, 4],
            stats[0, 2], stats[0, 3])


# speedup vs baseline: 1.0005x; 1.0005x over previous
import jax
import jax.numpy as jnp
from jax import lax
from jax.experimental import pallas as pl
from jax.experimental.pallas import tpu as pltpu

N = 100
NP = 128
IN_DIM = 512
HID = 512
H = 8
D = HID * H
FF = HID * 4
OUT_DIM = 256
K = 10
NEG = -1e30
F32 = jnp.float32

_CONTRACT_11 = (((1,), (1,)), ((), ()))
_CONTRACT_10 = (((1,), (0,)), ((), ()))

_NSLOT = 4
_B_W1, _B_W2, _B_QKV, _B_PROJ, _B_FF1, _B_FF2, _B_OUTP = 1, 9, 17, 41, 49, 53, 61
_NBLK = 62


def _dot(a, b, dims):
    return lax.dot_general(a, b, dims, preferred_element_type=F32)


def _ln(xa):
    mu = jnp.mean(xa, axis=1, keepdims=True)
    var = jnp.mean((xa - mu) ** 2, axis=1, keepdims=True)
    return (xa - mu) / jnp.sqrt(var + 1e-5)


def _graph_setup(x, stats_ref, cs):
    sims = _dot(x, x, _CONTRACT_11)
    col = lax.broadcasted_iota(jnp.int32, (NP, NP), 1)
    row = lax.broadcasted_iota(jnp.int32, (NP, NP), 0)
    valid_col = col < N
    valid_row = row < N

    selected = jnp.zeros((NP, NP), jnp.bool_)
    t_mat = jnp.zeros((NP, NP), F32)
    for t in range(K + 1):
        masked = jnp.where(valid_col & (~selected), sims, NEG)
        rowmax = jnp.max(masked, axis=1, keepdims=True)
        cand = jnp.where(masked == rowmax, col, NP)
        first = jnp.min(cand, axis=1, keepdims=True)
        newsel = col == first
        selected = selected | newsel
        if t > 0:
            t_mat = t_mat + newsel.astype(F32)
    t_mat = jnp.where(valid_row, t_mat, 0.0)
    cs[...] = t_mat.T + jnp.where((row == col) & valid_row, 1.0, 0.0)

    centrality = jnp.sum(sims, axis=1, keepdims=True)
    validr1 = lax.broadcasted_iota(jnp.int32, (NP, 1), 0) < N
    cmean = jnp.sum(centrality) / N
    cvar = jnp.sum(jnp.where(validr1, (centrality - cmean) ** 2, 0.0)) / (N - 1)
    cstd = jnp.sqrt(cvar)
    smean = jnp.sum(sims) / (N * N)
    degree = jnp.sum((sims > 0.5).astype(F32), axis=1, keepdims=True)
    s2 = _dot(sims, sims, _CONTRACT_10)
    tri = jnp.sum(s2 * sims, axis=1, keepdims=True)
    clus = tri / (degree * (degree - 1.0) + 1e-8)
    clustering = jnp.sum(jnp.where(validr1, clus, 0.0)) / N

    lane = lax.broadcasted_iota(jnp.int32, (8, 128), 1)
    rw = lax.broadcasted_iota(jnp.int32, (8, 128), 0)
    stats_ref[...] = jnp.where((rw == 0) & (lane == 0), cmean,
                     jnp.where((rw == 0) & (lane == 1), clustering,
                     jnp.where((rw == 0) & (lane == 2), smean,
                     jnp.where((rw == 0) & (lane == 3), cstd, 0.0))))


def _gat_attend(xp, asrc_ref, adst_ref, cmat, h):
    asr = asrc_ref[pl.ds(h, 1), :]
    adr = adst_ref[pl.ds(h, 1), :]
    a_s_row = _dot(asr, xp, _CONTRACT_11)
    a_d_col = _dot(xp, adr, _CONTRACT_11)
    e = a_s_row + a_d_col
    e = jnp.where(e >= 0, e, 0.2 * e)
    mask = cmat > 0.0
    em = jnp.where(mask, e, NEG)
    m = jnp.max(em, axis=1, keepdims=True)
    m = jnp.where(m > 0.5 * NEG, m, 0.0)
    ex = jnp.where(mask, jnp.exp(e - m), 0.0)
    z = jnp.sum(cmat * ex, axis=1, keepdims=True)
    alpha = ex / (z + 1e-16)
    out = _dot(cmat * alpha, xp, _CONTRACT_10)
    out = jnp.where(out > 0, out, jnp.exp(out) - 1.0)
    validr = lax.broadcasted_iota(jnp.int32, (NP, HID), 0) < N
    return jnp.where(validr, out, 0.0), alpha


def _attn_head(qkvs, asc, h):
    qh = qkvs[:, pl.ds(h * HID, HID)]
    kh = qkvs[:, pl.ds((h + H) * HID, HID)]
    vh = qkvs[:, pl.ds((h + 2 * H) * HID, HID)]
    logits = _dot(qh, kh, _CONTRACT_11) * (1.0 / jnp.sqrt(HID * 1.0))
    colmask = lax.broadcasted_iota(jnp.int32, (NP, NP), 1) < N
    logits = jnp.where(colmask, logits, NEG)
    m = jnp.max(logits, axis=1, keepdims=True)
    e = jnp.exp(logits - m)
    e = jnp.where(colmask, e, 0.0)
    att = e / jnp.sum(e, axis=1, keepdims=True)
    asc[:, pl.ds(h * HID, HID)] = _dot(att, vh, _CONTRACT_10)


def _edge_entropy(v, cmat, mask):
    vm = jnp.where(mask, v, NEG)
    mx = jnp.max(vm)
    e = jnp.where(mask, jnp.exp(v - mx), 0.0)
    s = jnp.sum(cmat * e)
    pr = e / s
    term = jnp.where(mask, pr * jnp.log(pr + 1e-8), 0.0)
    return -jnp.sum(cmat * term)


def _mega_body(x_ref, pe_ref, w0_ref, w1_ref, w2_ref,
               as0_ref, ad0_ref, as1_ref, ad1_ref, as2_ref, ad2_ref,
               win_ref, wout_ref, wf1_ref, wf2_ref, wo_ref,
               out_ref, stats_ref,
               g0s, gas, gbs, xpa, qkvs, asc, accs, x1s, fs, cs, vs,
               slots, sems):
    HH = HID // 2

    def copies(b):
        s = b % _NSLOT
        cs_ = []
        for k in range(2):
            rk = pl.ds(k * HH, HH)
            if b == 0:
                c = pltpu.make_async_copy(
                    w0_ref.at[rk, :], slots.at[s, rk, :], sems.at[s, k])
            elif b < _B_W2:
                c = pltpu.make_async_copy(
                    w1_ref.at[pl.ds((b - _B_W1) * HID + k * HH, HH), :],
                    slots.at[s, rk, :], sems.at[s, k])
            elif b < _B_QKV:
                c = pltpu.make_async_copy(
                    w2_ref.at[pl.ds((b - _B_W2) * HID + k * HH, HH), :],
                    slots.at[s, rk, :], sems.at[s, k])
            elif b < _B_PROJ:
                c = pltpu.make_async_copy(
                    win_ref.at[pl.ds((b - _B_QKV) * HID + k * HH, HH), :],
                    slots.at[s, rk, :], sems.at[s, k])
            elif b < _B_FF1:
                c = pltpu.make_async_copy(
                    wout_ref.at[pl.ds((b - _B_PROJ) * HID + k * HH, HH), :],
                    slots.at[s, rk, :], sems.at[s, k])
            elif b < _B_FF2:
                c = pltpu.make_async_copy(
                    wf1_ref.at[pl.ds((b - _B_FF1) * HID + k * HH, HH), :],
                    slots.at[s, rk, :], sems.at[s, k])
            elif b < _B_OUTP:
                c = pltpu.make_async_copy(
                    wf2_ref.at[pl.ds((b - _B_FF2) * HID + k * HH, HH), :],
                    slots.at[s, rk, pl.ds(0, FF)], sems.at[s, k])
            else:
                ho = OUT_DIM // 2
                c = pltpu.make_async_copy(
                    wo_ref.at[pl.ds(k * ho, ho), :],
                    slots.at[s, pl.ds(k * ho, ho), :], sems.at[s, k])
            cs_.append(c)
        return cs_

    def copy(b):
        class _Pair:
            def start(self):
                for c in copies(b):
                    c.start()
            def wait(self):
                for c in copies(b):
                    c.wait()
        return _Pair()

    for b in range(_NSLOT - 1):
        copy(b).start()

    x = x_ref[...]
    _graph_setup(x, stats_ref, cs)
    validg = lax.broadcasted_iota(jnp.int32, (NP, IN_DIM), 0) < N
    g0s[...] = jnp.where(validg, x + pe_ref[...], 0.0)

    def gat_attn_all(asrc_ref, adst_ref, gout, lidx):
        vsum = None
        for h in range(H):
            xph = xpa[:, pl.ds(h * HID, HID)]
            out, alpha = _gat_attend(xph, asrc_ref, adst_ref, cs[...], h)
            gout[:, pl.ds(h * HID, HID)] = out
            vsum = alpha * (1.0 / H) if vsum is None else vsum + alpha * (1.0 / H)
        vs[lidx] = vsum

    for b in range(_NBLK):
        copy(b).wait()
        if b + _NSLOT - 1 < _NBLK:
            copy(b + _NSLOT - 1).start()
        s = b % _NSLOT
        w = slots[s]
        if b == 0:
            xpa[...] = _dot(g0s[...], w, _CONTRACT_10)
            gat_attn_all(as0_ref, ad0_ref, gas, 0)
        elif b < _B_W2:
            r = b - _B_W1
            part = _dot(gas[:, pl.ds(r * HID, HID)], w, _CONTRACT_10)
            if r == 0:
                xpa[...] = part
            else:
                xpa[...] += part
            if r == H - 1:
                gat_attn_all(as1_ref, ad1_ref, gbs, 1)
        elif b < _B_QKV:
            r = b - _B_W2
            part = _dot(gbs[:, pl.ds(r * HID, HID)], w, _CONTRACT_10)
            if r == 0:
                xpa[...] = part
            else:
                xpa[...] += part
            if r == H - 1:
                gat_attn_all(as2_ref, ad2_ref, gas, 2)
        elif b < _B_PROJ:
            r = b - _B_QKV
            qkvs[:, pl.ds(r * HID, HID)] = _dot(gas[...], w, _CONTRACT_11)
            if r >= 2 * H + 1:
                _attn_head(qkvs, asc, r - (2 * H + 1))
        elif b < _B_FF1:
            jj = b - _B_PROJ
            if jj == 0:
                _attn_head(qkvs, asc, H - 1)
            accs[:, pl.ds(jj * HID, HID)] = _dot(asc[...], w, _CONTRACT_11)
            if jj == H - 1:
                x1s[...] = _ln(gas[...] + accs[...])
        elif b < _B_FF2:
            r = b - _B_FF1
            part = _dot(x1s[...], w, _CONTRACT_11)
            fs[:, pl.ds(r * HID, HID)] = jnp.maximum(part, 0.0)
        elif b < _B_OUTP:
            jj = b - _B_FF2
            accs[:, pl.ds(jj * HID, HID)] = _dot(fs[...], slots[s][:, :FF],
                                                 _CONTRACT_11)
        else:
            t = _ln(x1s[...] + accs[...])
            validr = lax.broadcasted_iota(jnp.int32, (NP, D), 0) < N
            tmean = jnp.sum(jnp.where(validr, t, 0.0), axis=0,
                            keepdims=True) / N
            out = _dot(tmean, slots[s][:OUT_DIM, :], _CONTRACT_11)
            out_ref[...] = jnp.broadcast_to(out, (8, OUT_DIM))

            cmat = cs[...]
            mask = cmat > 0.0
            ent = (_edge_entropy(vs[0], cmat, mask)
                   + _edge_entropy(vs[1], cmat, mask)
                   + _edge_entropy(vs[2], cmat, mask)) / 3.0
            lane = lax.broadcasted_iota(jnp.int32, (8, 128), 1)
            rw = lax.broadcasted_iota(jnp.int32, (8, 128), 0)
            stats_ref[...] = stats_ref[...] + jnp.where(
                (rw == 0) & (lane == 4), ent, 0.0)


def kernel(x, params):
    xp = jnp.pad(x, ((0, NP - N), (0, 0)))
    pep = jnp.pad(params['topo_pe'][:N, :IN_DIM], ((0, NP - N), (0, 0)))
    gat = params['gat']

    vm = pl.BlockSpec(memory_space=pltpu.VMEM)
    anym = pl.BlockSpec(memory_space=pl.ANY)
    outr, stats = pl.pallas_call(
        _mega_body,
        in_specs=[vm, vm, anym, anym, anym, vm, vm, vm, vm, vm, vm,
                  anym, anym, anym, anym, anym],
        out_specs=(vm, vm),
        out_shape=(
            jax.ShapeDtypeStruct((8, OUT_DIM), F32),
            jax.ShapeDtypeStruct((8, 128), F32),
        ),
        scratch_shapes=[
            pltpu.VMEM((NP, IN_DIM), F32),
            pltpu.VMEM((NP, D), F32),
            pltpu.VMEM((NP, D), F32),
            pltpu.VMEM((NP, D), F32),
            pltpu.VMEM((NP, 3 * D), F32),
            pltpu.VMEM((NP, D), F32),
            pltpu.VMEM((NP, D), F32),
            pltpu.VMEM((NP, D), F32),
            pltpu.VMEM((NP, FF), F32),
            pltpu.VMEM((NP, NP), F32),
            pltpu.VMEM((3, NP, NP), F32),
            pltpu.VMEM((_NSLOT, HID, D), F32),
            pltpu.SemaphoreType.DMA((_NSLOT, 2)),
        ],
    )(xp, pep, gat[0]['W'], gat[1]['W'], gat[2]['W'],
      gat[0]['a_src'], gat[0]['a_dst'], gat[1]['a_src'], gat[1]['a_dst'],
      gat[2]['a_src'], gat[2]['a_dst'],
      params['in_proj_w'], params['out_proj_w'],
      params['ff1_w'], params['ff2_w'], params['outp_w'])

    out = outr[0]
    return (out, stats[0, 0], stats[0, 1], stats[0, 4],
            stats[0, 2], stats[0, 3])


# speedup vs baseline: 1.0026x; 1.0022x over previous
import jax
import jax.numpy as jnp
from jax import lax
from jax.experimental import pallas as pl
from jax.experimental.pallas import tpu as pltpu

N = 100
NP = 128
IN_DIM = 512
HID = 512
H = 8
D = HID * H
FF = HID * 4
OUT_DIM = 256
K = 10
NEG = -1e30
F32 = jnp.float32

_CONTRACT_11 = (((1,), (1,)), ((), ()))
_CONTRACT_10 = (((1,), (0,)), ((), ()))

_NSLOT = 4
_B_W1, _B_W2, _B_QKV, _B_PROJ, _B_FF1, _B_FF2, _B_OUTP = 1, 9, 17, 41, 49, 53, 61
_NBLK = 62


def _dot(a, b, dims):
    return lax.dot_general(a, b, dims, preferred_element_type=F32)


def _ln(xa):
    mu = jnp.mean(xa, axis=1, keepdims=True)
    var = jnp.mean((xa - mu) ** 2, axis=1, keepdims=True)
    return (xa - mu) / jnp.sqrt(var + 1e-5)


def _graph_setup(x, stats_ref, cs):
    sims = _dot(x, x, _CONTRACT_11)
    col = lax.broadcasted_iota(jnp.int32, (NP, NP), 1)
    row = lax.broadcasted_iota(jnp.int32, (NP, NP), 0)
    valid_col = col < N
    valid_row = row < N

    selected = jnp.zeros((NP, NP), jnp.bool_)
    t_mat = jnp.zeros((NP, NP), F32)
    for t in range(K + 1):
        masked = jnp.where(valid_col & (~selected), sims, NEG)
        rowmax = jnp.max(masked, axis=1, keepdims=True)
        cand = jnp.where(masked == rowmax, col, NP)
        first = jnp.min(cand, axis=1, keepdims=True)
        newsel = col == first
        selected = selected | newsel
        if t > 0:
            t_mat = t_mat + newsel.astype(F32)
    t_mat = jnp.where(valid_row, t_mat, 0.0)
    cs[...] = t_mat.T + jnp.where((row == col) & valid_row, 1.0, 0.0)

    centrality = jnp.sum(sims, axis=1, keepdims=True)
    validr1 = lax.broadcasted_iota(jnp.int32, (NP, 1), 0) < N
    cmean = jnp.sum(centrality) / N
    cvar = jnp.sum(jnp.where(validr1, (centrality - cmean) ** 2, 0.0)) / (N - 1)
    cstd = jnp.sqrt(cvar)
    smean = jnp.sum(sims) / (N * N)
    degree = jnp.sum((sims > 0.5).astype(F32), axis=1, keepdims=True)
    s2 = _dot(sims, sims, _CONTRACT_10)
    tri = jnp.sum(s2 * sims, axis=1, keepdims=True)
    clus = tri / (degree * (degree - 1.0) + 1e-8)
    clustering = jnp.sum(jnp.where(validr1, clus, 0.0)) / N

    lane = lax.broadcasted_iota(jnp.int32, (8, 128), 1)
    rw = lax.broadcasted_iota(jnp.int32, (8, 128), 0)
    stats_ref[...] = jnp.where((rw == 0) & (lane == 0), cmean,
                     jnp.where((rw == 0) & (lane == 1), clustering,
                     jnp.where((rw == 0) & (lane == 2), smean,
                     jnp.where((rw == 0) & (lane == 3), cstd, 0.0))))


def _gat_attend(xp, asrc_ref, adst_ref, cmat, h):
    asr = asrc_ref[pl.ds(h, 1), :]
    adr = adst_ref[pl.ds(h, 1), :]
    a_s_row = _dot(asr, xp, _CONTRACT_11)
    a_d_col = _dot(xp, adr, _CONTRACT_11)
    e = a_s_row + a_d_col
    e = jnp.where(e >= 0, e, 0.2 * e)
    mask = cmat > 0.0
    em = jnp.where(mask, e, NEG)
    m = jnp.max(em, axis=1, keepdims=True)
    m = jnp.where(m > 0.5 * NEG, m, 0.0)
    ex = jnp.where(mask, jnp.exp(e - m), 0.0)
    z = jnp.sum(cmat * ex, axis=1, keepdims=True)
    alpha = ex / (z + 1e-16)
    out = _dot(cmat * alpha, xp, _CONTRACT_10)
    out = jnp.where(out > 0, out, jnp.exp(out) - 1.0)
    validr = lax.broadcasted_iota(jnp.int32, (NP, HID), 0) < N
    return jnp.where(validr, out, 0.0), alpha


def _attn_head(qkvs, asc, h):
    qh = qkvs[:, pl.ds(h * HID, HID)]
    kh = qkvs[:, pl.ds((h + H) * HID, HID)]
    vh = qkvs[:, pl.ds((h + 2 * H) * HID, HID)]
    logits = _dot(qh, kh, _CONTRACT_11) * (1.0 / jnp.sqrt(HID * 1.0))
    colmask = lax.broadcasted_iota(jnp.int32, (NP, NP), 1) < N
    logits = jnp.where(colmask, logits, NEG)
    m = jnp.max(logits, axis=1, keepdims=True)
    e = jnp.exp(logits - m)
    e = jnp.where(colmask, e, 0.0)
    att = e / jnp.sum(e, axis=1, keepdims=True)
    asc[:, pl.ds(h * HID, HID)] = _dot(att, vh, _CONTRACT_10)


def _edge_entropy(v, cmat, mask):
    vm = jnp.where(mask, v, NEG)
    mx = jnp.max(vm)
    e = jnp.where(mask, jnp.exp(v - mx), 0.0)
    s = jnp.sum(cmat * e)
    pr = e / s
    term = jnp.where(mask, pr * jnp.log(pr + 1e-8), 0.0)
    return -jnp.sum(cmat * term)


def _mega_body(x_ref, pe_ref, w0_ref, w1_ref, w2_ref,
               as0_ref, ad0_ref, as1_ref, ad1_ref, as2_ref, ad2_ref,
               win_ref, wout_ref, wf1_ref, wf2_ref, wo_ref,
               out_ref, stats_ref,
               g0s, gas, gbs, xpa, qkvs, asc, accs, x1s, fs, cs, vs,
               slots, sems):
    def copy(b):
        s = b % _NSLOT
        if b == 0:
            return pltpu.make_async_copy(w0_ref, slots.at[s], sems.at[s])
        if b < _B_W2:
            src = w1_ref.at[pl.ds((b - _B_W1) * HID, HID), :]
        elif b < _B_QKV:
            src = w2_ref.at[pl.ds((b - _B_W2) * HID, HID), :]
        elif b < _B_PROJ:
            src = win_ref.at[pl.ds((b - _B_QKV) * HID, HID), :]
        elif b < _B_FF1:
            src = wout_ref.at[pl.ds((b - _B_PROJ) * HID, HID), :]
        elif b < _B_FF2:
            src = wf1_ref.at[pl.ds((b - _B_FF1) * HID, HID), :]
        elif b < _B_OUTP:
            return pltpu.make_async_copy(
                wf2_ref.at[pl.ds((b - _B_FF2) * HID, HID), :],
                slots.at[s, :, pl.ds(0, FF)], sems.at[s])
        else:
            return pltpu.make_async_copy(
                wo_ref, slots.at[s, pl.ds(0, OUT_DIM), :], sems.at[s])
        return pltpu.make_async_copy(src, slots.at[s], sems.at[s])

    for b in range(_NSLOT - 1):
        copy(b).start()

    x = x_ref[...]
    _graph_setup(x, stats_ref, cs)
    validg = lax.broadcasted_iota(jnp.int32, (NP, IN_DIM), 0) < N
    g0s[...] = jnp.where(validg, x + pe_ref[...], 0.0)

    def gat_attn_all(asrc_ref, adst_ref, gout, lidx):
        vsum = None
        for h in range(H):
            xph = xpa[:, pl.ds(h * HID, HID)]
            out, alpha = _gat_attend(xph, asrc_ref, adst_ref, cs[...], h)
            gout[:, pl.ds(h * HID, HID)] = out
            vsum = alpha * (1.0 / H) if vsum is None else vsum + alpha * (1.0 / H)
        vs[lidx] = vsum

    for b in range(_NBLK):
        copy(b).wait()
        if b + _NSLOT - 1 < _NBLK:
            copy(b + _NSLOT - 1).start()
        s = b % _NSLOT
        w = slots[s]
        if b == 0:
            xpa[...] = _dot(g0s[...], w, _CONTRACT_10)
            gat_attn_all(as0_ref, ad0_ref, gas, 0)
        elif b < _B_W2:
            r = b - _B_W1
            part = _dot(gas[:, pl.ds(r * HID, HID)], w, _CONTRACT_10)
            if r == 0:
                xpa[...] = part
            else:
                xpa[...] += part
            if r == H - 1:
                gat_attn_all(as1_ref, ad1_ref, gbs, 1)
        elif b < _B_QKV:
            r = b - _B_W2
            part = _dot(gbs[:, pl.ds(r * HID, HID)], w, _CONTRACT_10)
            if r == 0:
                xpa[...] = part
            else:
                xpa[...] += part
            if r == H - 1:
                gat_attn_all(as2_ref, ad2_ref, gas, 2)
        elif b < _B_PROJ:
            r = b - _B_QKV
            qkvs[:, pl.ds(r * HID, HID)] = _dot(gas[...], w, _CONTRACT_11)
            if r >= 2 * H + 1:
                _attn_head(qkvs, asc, r - (2 * H + 1))
        elif b < _B_FF1:
            jj = b - _B_PROJ
            if jj == 0:
                _attn_head(qkvs, asc, H - 1)
            accs[:, pl.ds(jj * HID, HID)] = _dot(asc[...], w, _CONTRACT_11)
            if jj == H - 1:
                x1s[...] = _ln(gas[...] + accs[...])
        elif b < _B_FF2:
            r = b - _B_FF1
            part = _dot(x1s[...], w, _CONTRACT_11)
            fs[:, pl.ds(r * HID, HID)] = jnp.maximum(part, 0.0)
        elif b < _B_OUTP:
            jj = b - _B_FF2
            accs[:, pl.ds(jj * HID, HID)] = _dot(fs[...], slots[s][:, :FF],
                                                 _CONTRACT_11)
        else:
            t = _ln(x1s[...] + accs[...])
            validr = lax.broadcasted_iota(jnp.int32, (NP, D), 0) < N
            tmean = jnp.sum(jnp.where(validr, t, 0.0), axis=0,
                            keepdims=True) / N
            out = _dot(tmean, slots[s][:OUT_DIM, :], _CONTRACT_11)
            out_ref[...] = jnp.broadcast_to(out, (8, OUT_DIM))

            cmat = cs[...]
            mask = cmat > 0.0
            ent = (_edge_entropy(vs[0], cmat, mask)
                   + _edge_entropy(vs[1], cmat, mask)
                   + _edge_entropy(vs[2], cmat, mask)) / 3.0
            lane = lax.broadcasted_iota(jnp.int32, (8, 128), 1)
            rw = lax.broadcasted_iota(jnp.int32, (8, 128), 0)
            stats_ref[...] = stats_ref[...] + jnp.where(
                (rw == 0) & (lane == 4), ent, 0.0)


def kernel(x, params):
    xp = jnp.pad(x, ((0, NP - N), (0, 0)))
    pep = jnp.pad(params['topo_pe'][:N, :IN_DIM], ((0, NP - N), (0, 0)))
    gat = params['gat']

    vm = pl.BlockSpec(memory_space=pltpu.VMEM)
    anym = pl.BlockSpec(memory_space=pl.ANY)
    outr, stats = pl.pallas_call(
        _mega_body,
        in_specs=[vm, vm, anym, anym, anym, vm, vm, vm, vm, vm, vm,
                  anym, anym, anym, anym, anym],
        out_specs=(vm, vm),
        out_shape=(
            jax.ShapeDtypeStruct((8, OUT_DIM), F32),
            jax.ShapeDtypeStruct((8, 128), F32),
        ),
        scratch_shapes=[
            pltpu.VMEM((NP, IN_DIM), F32),
            pltpu.VMEM((NP, D), F32),
            pltpu.VMEM((NP, D), F32),
            pltpu.VMEM((NP, D), F32),
            pltpu.VMEM((NP, 3 * D), F32),
            pltpu.VMEM((NP, D), F32),
            pltpu.VMEM((NP, D), F32),
            pltpu.VMEM((NP, D), F32),
            pltpu.VMEM((NP, FF), F32),
            pltpu.VMEM((NP, NP), F32),
            pltpu.VMEM((3, NP, NP), F32),
            pltpu.VMEM((_NSLOT, HID, D), F32),
            pltpu.SemaphoreType.DMA((_NSLOT,)),
        ],
    )(xp, pep, gat[0]['W'], gat[1]['W'], gat[2]['W'],
      gat[0]['a_src'], gat[0]['a_dst'], gat[1]['a_src'], gat[1]['a_dst'],
      gat[2]['a_src'], gat[2]['a_dst'],
      params['in_proj_w'], params['out_proj_w'],
      params['ff1_w'], params['ff2_w'], params['outp_w'])

    out = outr[0]
    return (out, stats[0, 0], stats[0, 1], stats[0, 4],
            stats[0, 2], stats[0, 3])
